# baseline (speedup 1.0000x reference)
import functools

import jax
import jax.numpy as jnp
from jax import lax
from jax.experimental import pallas as pl
from jax.experimental.pallas import tpu as pltpu
from jax.experimental.pallas import tpu_sc as plsc

_B = 1024
_L = 200
_D = 64
_SPLITS = ((0, 104), (104, 96))
_NC, _NS = 2, 16
_NW = _NC * _NS
_RPW = _B // _NW
_LANES = 16


@functools.partial(
    pl.kernel,
    out_type=jax.ShapeDtypeStruct((_B, _L, _D), jnp.float32),
    mesh=plsc.VectorSubcoreMesh(core_axis_name="c", subcore_axis_name="s",
                                num_cores=_NC),
    scratch_types=(
        [pltpu.VMEM((_L,), jnp.int32)] * 2
        + [pltpu.VMEM((_L, 2 * _D), jnp.float32)] * 2
        + [pltpu.VMEM((_L, _D), jnp.float32)] * 2
        + [pltpu.VMEM((_L // 2, 2 * _D), jnp.float32)]
        + [pltpu.SemaphoreType.DMA] * 6
    ),
)
def _embed_kernel(ids_hbm, t128_hbm, wpe2_hbm, out_hbm, *refs):
    idx_v = refs[0:2]
    rows_v = refs[2:4]
    sum_v = refs[4:6]
    wpe2_v = refs[6]
    idx_sem = refs[7:9]
    g_sem = refs[9:11]
    out_sem = refs[11:13]

    wid = lax.axis_index("s") * _NC + lax.axis_index("c")
    base = wid * _RPW

    pltpu.sync_copy(wpe2_hbm, wpe2_v)

    def row_of(i):
        return base + i

    def issue_ids(i, b):
        pltpu.async_copy(ids_hbm.at[pl.ds(row_of(i) * _L, _L)], idx_v[b],
                         idx_sem[b])

    def drain_ids(i, b):
        pltpu.make_async_copy(ids_hbm.at[pl.ds(row_of(i) * _L, _L)],
                              idx_v[b], idx_sem[b]).wait()

    def issue_gathers(b):
        for off, size in _SPLITS:
            pltpu.async_copy(t128_hbm.at[idx_v[b].at[pl.ds(off, size)]],
                             rows_v[b].at[pl.ds(off, size)], g_sem[b])

    def drain_gathers(b):
        for off, size in _SPLITS:
            pltpu.make_async_copy(t128_hbm.at[pl.ds(0, size)],
                                  rows_v[b].at[pl.ds(off, size)],
                                  g_sem[b]).wait()

    def add_wpe(b):
        @pl.loop(0, _L // 2, unroll=8)
        def _add(l2):
            for h in range(2):
                for c in range(_D // _LANES):
                    sl = pl.ds(c * _LANES, _LANES)
                    wsl = pl.ds(h * _D + c * _LANES, _LANES)
                    sum_v[b][2 * l2 + h, sl] = (
                        rows_v[b][2 * l2 + h, sl] + wpe2_v[l2, wsl])

    def issue_out(i, b):
        pltpu.async_copy(sum_v[b], out_hbm.at[row_of(i)], out_sem[b])

    def drain_out(i, b):
        pltpu.make_async_copy(sum_v[b], out_hbm.at[row_of(i)],
                              out_sem[b]).wait()

    def step(i, b):
        bp = 1 - b
        if 1 <= i <= _RPW:
            drain_gathers(bp)
            add_wpe(bp)
        if 2 <= i <= _RPW + 1:
            drain_out(i - 2, b)
        if 1 <= i <= _RPW:
            issue_out(i - 1, bp)
        if i + 1 <= _RPW - 1:
            issue_ids(i + 1, bp)
        if i <= _RPW - 1:
            drain_ids(i, b)
            issue_gathers(b)

    issue_ids(0, 0)
    step(0, 0)
    step(1, 1)

    @pl.loop(2, 30, step=2)
    def _steady(j):
        def dstep(i, b):
            bp = 1 - b
            drain_gathers(bp)
            add_wpe(bp)
            pltpu.make_async_copy(sum_v[b], out_hbm.at[i - 2 + base],
                                  out_sem[b]).wait()
            pltpu.async_copy(sum_v[bp], out_hbm.at[i - 1 + base],
                             out_sem[bp])
            pltpu.async_copy(ids_hbm.at[pl.ds((i + 1 + base) * _L, _L)],
                             idx_v[bp], idx_sem[bp])
            pltpu.make_async_copy(ids_hbm.at[pl.ds((i + base) * _L, _L)],
                                  idx_v[b], idx_sem[b]).wait()
            issue_gathers(b)

        dstep(j, 0)
        dstep(j + 1, 1)

    for i in range(30, _RPW + 2):
        step(i, i % 2)


def kernel(input_ids, wte_table, wpe_table):
    t128 = jnp.pad(wte_table, ((0, 0), (0, _D)))
    ids = input_ids.reshape(-1).astype(jnp.int32)
    wpe2 = wpe_table[:_L].reshape(_L // 2, 2 * _D)
    return _embed_kernel(ids, t128, wpe2)


# speedup vs baseline: 1.0231x; 1.0231x over previous
import functools

import jax
import jax.numpy as jnp
from jax import lax
from jax.experimental import pallas as pl
from jax.experimental.pallas import tpu as pltpu
from jax.experimental.pallas import tpu_sc as plsc

_B = 1024
_L = 200
_D = 64
_CHUNK = 100
_CPR = _L // _CHUNK
_NC, _NS = 2, 16
_NW = _NC * _NS
_RPW = _B // _NW
_G = 4


@functools.partial(
    pl.kernel,
    out_type=jax.ShapeDtypeStruct((_B, _CPR, _CHUNK, _D), jnp.float32),
    mesh=plsc.VectorSubcoreMesh(core_axis_name="c", subcore_axis_name="s",
                                num_cores=_NC),
    scratch_types=(
        [pltpu.VMEM((_G, _CPR, _CHUNK), jnp.int32),
         pltpu.VMEM((_G, _CPR, _CHUNK, _D), jnp.float32),
         pltpu.VMEM_SHARED((_CPR, _CHUNK, _D), jnp.float32)]
        + [pltpu.SemaphoreType.DMA] * (4 * _G)
    ),
    compiler_params=pltpu.CompilerParams(use_tc_tiling_on_sc=False),
)
def _embed_kernel(ids_hbm, wte_hbm, wpe_hbm, out_hbm, idx_v, rows_v,
                  wpe_sh, *sems):
    idx_sem = sems[0:_G]
    init_sem = sems[_G:2 * _G]
    g_sem = sems[2 * _G:3 * _G]
    out_sem = sems[3 * _G:4 * _G]

    cid = lax.axis_index("c")
    sid = lax.axis_index("s")
    wid = sid * _NC + cid
    base = wid * _RPW

    @pl.when(sid == 0)
    def _stage_wpe():
        for c in range(_CPR):
            pltpu.sync_copy(wpe_hbm.at[pl.ds(c * _CHUNK, _CHUNK)],
                            rows_v.at[0, c])
            pltpu.sync_copy(rows_v.at[0, c], wpe_sh.at[c])

    plsc.subcore_barrier()

    @pl.loop(0, _RPW, step=_G)
    def _group(g):
        ins = []
        for r in range(_G):
            row = base + g + r
            d_idx = pltpu.async_copy(ids_hbm.at[row], idx_v.at[r],
                                     idx_sem[r])
            d_init = pltpu.async_copy(wpe_sh, rows_v.at[r], init_sem[r])
            ins.append((d_idx, d_init))

        gathers = []
        for r in range(_G):
            ins[r][0].wait()
            ins[r][1].wait()
            for c in range(_CPR):
                gathers.append(
                    pltpu.async_copy(wte_hbm.at[idx_v.at[r, c]],
                                     rows_v.at[r, c], g_sem[r], add=True))

        outs = []
        for r in range(_G):
            for c in range(_CPR):
                gathers[_CPR * r + c].wait()
            outs.append(pltpu.async_copy(rows_v.at[r],
                                         out_hbm.at[base + g + r],
                                         out_sem[r]))

        for d in outs:
            d.wait()


def kernel(input_ids, wte_table, wpe_table):
    ids = input_ids.reshape(_B, _CPR, _CHUNK).astype(jnp.int32)
    out = _embed_kernel(ids, wte_table, wpe_table)
    return out.reshape(_B, _L, _D)
